# baseline (speedup 1.0000x reference)
import functools
import math

import jax
import jax.numpy as jnp
from jax import lax
from jax.experimental import pallas as pl
from jax.experimental.pallas import tpu as pltpu
from jax.experimental.pallas import tpu_sc as plsc

D_MODEL = 64
SCALE = math.sqrt(D_MODEL)
NBUF = 2
BLK = 128


@functools.lru_cache(maxsize=None)
def _build(n_i0, n_i1, D):
    NC, NS = 2, 16
    NW = NC * NS
    n_b0 = n_i0 // BLK
    n_blocks = n_i1 * n_b0
    assert n_blocks % NW == 0
    blocks_per_w = n_blocks // NW
    GD = D // 8

    mesh = plsc.VectorSubcoreMesh(
        core_axis_name="c", subcore_axis_name="s", num_cores=NC, num_subcores=NS
    )

    @functools.partial(
        pl.kernel,
        mesh=mesh,
        out_type=jax.ShapeDtypeStruct((n_i1, GD, n_b0, 8 * BLK), jnp.float32),
        scratch_types=[
            pltpu.VMEM((blocks_per_w, BLK), jnp.int32),
            pltpu.VMEM((NBUF, BLK, D), jnp.float32),
            pltpu.VMEM((NBUF, GD, 8 * BLK), jnp.float32),
            pltpu.SemaphoreType.DMA((NBUF,)),
            pltpu.SemaphoreType.DMA((NBUF,)),
        ],
        compiler_params=pltpu.CompilerParams(
            use_tc_tiling_on_sc=False, needs_layout_passes=False
        ),
    )
    def emb_kernel(idx_hbm, table_hbm, out_hbm, idx_v, gbuf, sbuf, gsem, ssem):
        wid = lax.axis_index("s") * NC + lax.axis_index("c")
        base_b = wid * blocks_per_w
        pltpu.sync_copy(idx_hbm.at[pl.ds(base_b, blocks_per_w)], idx_v)
        l_iota = lax.iota(jnp.int32, 16)

        def gather_start(j, b):
            pltpu.async_copy(
                table_hbm.at[idx_v.at[j]], gbuf.at[b], gsem.at[b]
            )

        def gather_wait(b):
            pltpu.make_async_copy(
                table_hbm.at[idx_v.at[0]], gbuf.at[b], gsem.at[b]
            ).wait()

        def transpose_scale(gb, sb):
            @pl.loop(0, D)
            def _feat(i2):
                g = i2 // 8
                s = i2 % 8
                col = s * BLK
                for k in range(BLK // 16):
                    v = plsc.load_gather(
                        gbuf.at[gb], [jnp.full((16,), k * 16, jnp.int32) + l_iota,
                                      jnp.full((16,), i2, jnp.int32)]
                    )
                    sbuf[sb, g, pl.ds(col + k * 16, 16)] = v * SCALE

        def store_start(j, b):
            B = base_b + j
            i1 = B // n_b0
            b0 = B % n_b0
            pltpu.async_copy(
                sbuf.at[b],
                out_hbm.at[i1, :, b0, :],
                ssem.at[b],
            )

        def store_wait(b):
            pltpu.make_async_copy(
                sbuf.at[b], out_hbm.at[0, :, 0, :], ssem.at[b]
            ).wait()

        for b in range(NBUF):
            gather_start(b, b)

        for b in range(NBUF):
            gather_wait(b)
            transpose_scale(b, b)
            store_start(b, b)
            gather_start(b + NBUF, b)

        @pl.loop(NBUF, blocks_per_w - NBUF, step=NBUF)
        def _main(j0):
            for b in range(NBUF):
                j = j0 + b
                gather_wait(b)
                store_wait(b)
                transpose_scale(b, b)
                store_start(j, b)
                gather_start(j + NBUF, b)

        for b in range(NBUF):
            j = blocks_per_w - NBUF + b
            gather_wait(b)
            store_wait(b)
            transpose_scale(b, b)
            store_start(j, b)

        for b in range(NBUF):
            store_wait(b)

    return emb_kernel


def kernel(x, table):
    n_i0, n_i1 = x.shape
    D = table.shape[1]
    n_b0 = n_i0 // BLK
    idx = x.T.reshape(n_i1 * n_b0, BLK).astype(jnp.int32)
    out = _build(n_i0, n_i1, D)(idx, table)
    out = out.reshape(n_i1, D // 8, n_b0, 8, BLK)
    out = out.transpose(2, 4, 0, 1, 3)
    return out.reshape(n_i0, n_i1, D)


# speedup vs baseline: 1.3864x; 1.3864x over previous
import functools
import math

import jax
import jax.numpy as jnp
from jax import lax
from jax.experimental import pallas as pl
from jax.experimental.pallas import tpu as pltpu
from jax.experimental.pallas import tpu_sc as plsc

D_MODEL = 64
SCALE = math.sqrt(D_MODEL)
NBUF = 2
BLK = 128


@functools.lru_cache(maxsize=None)
def _build(n_i0, n_i1, D):
    NC, NS = 2, 16
    NW = NC * NS
    n_b0 = n_i0 // BLK
    n_blocks = n_i1 * n_b0
    assert n_blocks % NW == 0
    blocks_per_w = n_blocks // NW
    GD = D // 8

    mesh = plsc.VectorSubcoreMesh(
        core_axis_name="c", subcore_axis_name="s", num_cores=NC, num_subcores=NS
    )

    @functools.partial(
        pl.kernel,
        mesh=mesh,
        out_type=jax.ShapeDtypeStruct((n_i1, GD, n_b0, 8 * BLK), jnp.float32),
        scratch_types=[
            pltpu.VMEM((blocks_per_w, BLK), jnp.int32),
            pltpu.VMEM((NBUF, BLK, D), jnp.float32),
            pltpu.VMEM((NBUF, GD, 8 * BLK), jnp.float32),
            pltpu.SemaphoreType.DMA((NBUF,)),
            pltpu.SemaphoreType.DMA((NBUF,)),
        ],
        compiler_params=pltpu.CompilerParams(
            use_tc_tiling_on_sc=False, needs_layout_passes=False
        ),
    )
    def emb_kernel(idx_hbm, table_hbm, out_hbm, idx_v, gbuf, sbuf, gsem, ssem):
        wid = lax.axis_index("s") * NC + lax.axis_index("c")
        base_b = wid * blocks_per_w
        pltpu.sync_copy(idx_hbm.at[pl.ds(base_b, blocks_per_w)], idx_v)
        l_iota = lax.iota(jnp.int32, 16)

        def gather_start(j, b):
            pltpu.async_copy(
                table_hbm.at[idx_v.at[j]], gbuf.at[b], gsem.at[b]
            )

        def gather_wait(b):
            pltpu.make_async_copy(
                table_hbm.at[idx_v.at[0]], gbuf.at[b], gsem.at[b]
            ).wait()

        def transpose_scale(gb, sb):
            @plsc.parallel_loop(0, D, unroll=2)
            def _feat(i2):
                g = i2 // 8
                s = i2 % 8
                col = s * BLK
                for k in range(BLK // 16):
                    v = plsc.load_gather(
                        gbuf.at[gb], [jnp.full((16,), k * 16, jnp.int32) + l_iota,
                                      jnp.full((16,), i2, jnp.int32)]
                    )
                    sbuf[sb, g, pl.ds(col + k * 16, 16)] = v * SCALE

        def store_start(j, b):
            B = base_b + j
            i1 = B // n_b0
            b0 = B % n_b0
            pltpu.async_copy(
                sbuf.at[b],
                out_hbm.at[i1, :, b0, :],
                ssem.at[b],
            )

        def store_wait(b):
            pltpu.make_async_copy(
                sbuf.at[b], out_hbm.at[0, :, 0, :], ssem.at[b]
            ).wait()

        for b in range(NBUF):
            gather_start(b, b)

        for b in range(NBUF):
            gather_wait(b)
            transpose_scale(b, b)
            store_start(b, b)
            gather_start(b + NBUF, b)

        @pl.loop(NBUF, blocks_per_w - NBUF, step=NBUF)
        def _main(j0):
            for b in range(NBUF):
                j = j0 + b
                gather_wait(b)
                store_wait(b)
                transpose_scale(b, b)
                store_start(j, b)
                gather_start(j + NBUF, b)

        for b in range(NBUF):
            j = blocks_per_w - NBUF + b
            gather_wait(b)
            store_wait(b)
            transpose_scale(b, b)
            store_start(j, b)

        for b in range(NBUF):
            store_wait(b)

    return emb_kernel


def kernel(x, table):
    n_i0, n_i1 = x.shape
    D = table.shape[1]
    n_b0 = n_i0 // BLK
    idx = x.T.reshape(n_i1 * n_b0, BLK).astype(jnp.int32)
    out = _build(n_i0, n_i1, D)(idx, table)
    out = out.reshape(n_i1, D // 8, n_b0, 8, BLK)
    out = out.transpose(2, 4, 0, 1, 3)
    return out.reshape(n_i0, n_i1, D)
